# baseline (speedup 1.0000x reference)
import functools

import jax
import jax.numpy as jnp
from jax import lax
from jax.experimental import pallas as pl
from jax.experimental.pallas import tpu as pltpu
from jax.experimental.pallas import tpu_sc as plsc

INPUT_DIM = 257
OUTPUT_DIM = 32
MAX_POS = (INPUT_DIM - 1) // 2


def _make_sc_kernel(S, D):
    NC, NS = 2, 16
    NW = NC * NS
    q_per_worker = S // NW
    n_vtiles = S // 128
    n_band = 3
    n_const = n_vtiles - n_band
    t0_max = n_vtiles - n_band

    mesh = plsc.VectorSubcoreMesh(
        core_axis_name="c", subcore_axis_name="s", num_cores=NC, num_subcores=NS
    )

    @functools.partial(
        pl.kernel,
        mesh=mesh,
        out_type=jax.ShapeDtypeStruct((S, D, S), jnp.float32),
        scratch_types=[
            pltpu.VMEM((INPUT_DIM * D,), jnp.float32),
            pltpu.VMEM((D, 2 * n_band * 128), jnp.float32),
            pltpu.VMEM_SHARED((D, 1024), jnp.float32),
            pltpu.VMEM_SHARED((D, 1024), jnp.float32),
            pltpu.SemaphoreType.DMA,
            pltpu.SemaphoreType.DMA,
            pltpu.SemaphoreType.DMA,
        ],
        compiler_params=pltpu.CompilerParams(
            use_tc_tiling_on_sc=True, needs_layout_passes=False
        ),
    )
    def k(table_hbm, out_hbm, table_v, band_v, cl_sh, cr_sh, sem_c, sem_b0, sem_b1):
        c = lax.axis_index("c")
        s = lax.axis_index("s")
        wid = s * NC + c
        q0 = wid * q_per_worker

        pltpu.sync_copy(table_hbm, table_v)
        lane = lax.iota(jnp.int32, 16)

        def drain_bytes(sem, widths):
            for w in widths:
                pltpu.make_async_copy(
                    out_hbm.at[0, :, pl.ds(0, w)],
                    cl_sh.at[:, pl.ds(0, w)],
                    sem,
                ).wait()

        const_drain_widths = [1024, 512, 128]
        band_drain_widths = [n_band * 128]

        def t0_of(start):
            return jnp.clip((S - MAX_POS - 128 - start) // 128, 0, t0_max)

        @pl.when(s == 0)
        def _():
            for d in range(D):
                left = plsc.load_gather(table_v, [jnp.full((16,), d, jnp.int32)])
                right = plsc.load_gather(
                    table_v, [jnp.full((16,), (INPUT_DIM - 1) * D + d, jnp.int32)]
                )
                for g in range(16):
                    band_v[d, pl.ds(16 * g, 16)] = left
                    band_v[d, pl.ds(256 + 16 * g, 16)] = right
            for rep in range(4):
                pltpu.sync_copy(
                    band_v.at[:, pl.ds(0, 256)], cl_sh.at[:, pl.ds(256 * rep, 256)]
                )
                pltpu.sync_copy(
                    band_v.at[:, pl.ds(256, 256)], cr_sh.at[:, pl.ds(256 * rep, 256)]
                )

        plsc.subcore_barrier()

        def qloop(i, carry):
            q = q0 + i
            start = (S - 1) - q
            t0 = t0_of(start)
            t0_prev = t0_of(start + 2)
            par = lax.rem(i, 2)
            boff = par * (n_band * 128)

            @pl.when(jnp.logical_and(i >= 2, par == 0))
            def _():
                drain_bytes(sem_b0, band_drain_widths)

            @pl.when(jnp.logical_and(i >= 2, par == 1))
            def _():
                drain_bytes(sem_b1, band_drain_widths)

            full = jnp.logical_or(i < 2, t0 != t0_prev)
            rel = S - 2 - start - 128 * t0
            g_lo = jnp.where(full, 0, jnp.clip((rel - MAX_POS) // 16, 0, n_band * 8))
            g_hi = jnp.where(
                full, n_band * 8, jnp.clip((rel + MAX_POS) // 16 + 1, 0, n_band * 8)
            )

            def gbody(g, carry2):
                kk = start + 128 * t0 + 16 * g + lane
                ids = jnp.clip(kk - (S - 1), -MAX_POS, MAX_POS) + MAX_POS
                base = ids * D
                for d in range(D):
                    band_v[d, pl.ds(boff + 16 * g, 16)] = plsc.load_gather(
                        table_v, [base + d]
                    )
                return carry2

            lax.fori_loop(g_lo, g_hi, gbody, 0)

            @pl.when(i >= 1)
            def _():
                drain_bytes(sem_c, const_drain_widths)

            band_src = band_v.at[:, pl.ds(boff, n_band * 128)]

            @pl.when(par == 0)
            def _():
                pltpu.async_copy(
                    band_src, out_hbm.at[q, :, pl.ds(128 * t0, n_band * 128)], sem_b0
                )

            @pl.when(par == 1)
            def _():
                pltpu.async_copy(
                    band_src, out_hbm.at[q, :, pl.ds(128 * t0, n_band * 128)], sem_b1
                )

            b8 = lax.rem(t0 // 8, 2)
            b4 = lax.rem(t0 // 4, 2)
            b2 = lax.rem(t0 // 2, 2)
            b1 = lax.rem(t0, 2)
            p1024 = 128 * t0 - 1024
            p512 = 128 * t0 - 1024 * b8 - 512
            p256 = 128 * t0 - 1024 * b8 - 512 * b4 - 256
            p128 = 128 * t0 - 1024 * b8 - 512 * b4 - 256 * b2 - 128

            @pl.when(b8 == 1)
            def _():
                pltpu.async_copy(
                    cl_sh, out_hbm.at[q, :, pl.ds(p1024, 1024)], sem_c
                )

            @pl.when(b4 == 1)
            def _():
                pltpu.async_copy(
                    cl_sh.at[:, pl.ds(0, 512)],
                    out_hbm.at[q, :, pl.ds(p512, 512)],
                    sem_c,
                )

            @pl.when(b2 == 1)
            def _():
                pltpu.async_copy(
                    cl_sh.at[:, pl.ds(0, 256)],
                    out_hbm.at[q, :, pl.ds(p256, 256)],
                    sem_c,
                )

            @pl.when(b1 == 1)
            def _():
                pltpu.async_copy(
                    cl_sh.at[:, pl.ds(0, 128)],
                    out_hbm.at[q, :, pl.ds(p128, 128)],
                    sem_c,
                )

            m = t0_max - t0
            m8 = lax.rem(m // 8, 2)
            m4 = lax.rem(m // 4, 2)
            m2 = lax.rem(m // 2, 2)
            m1 = lax.rem(m, 2)
            r0 = 128 * (t0 + n_band)
            r1024 = r0
            r512 = r0 + 1024 * m8
            r256 = r512 + 512 * m4
            r128 = r256 + 256 * m2

            @pl.when(m8 == 1)
            def _():
                pltpu.async_copy(
                    cr_sh, out_hbm.at[q, :, pl.ds(r1024, 1024)], sem_c
                )

            @pl.when(m4 == 1)
            def _():
                pltpu.async_copy(
                    cr_sh.at[:, pl.ds(0, 512)],
                    out_hbm.at[q, :, pl.ds(r512, 512)],
                    sem_c,
                )

            @pl.when(m2 == 1)
            def _():
                pltpu.async_copy(
                    cr_sh.at[:, pl.ds(0, 256)],
                    out_hbm.at[q, :, pl.ds(r256, 256)],
                    sem_c,
                )

            @pl.when(m1 == 1)
            def _():
                pltpu.async_copy(
                    cr_sh.at[:, pl.ds(0, 128)],
                    out_hbm.at[q, :, pl.ds(r128, 128)],
                    sem_c,
                )

            return carry

        lax.fori_loop(0, q_per_worker, qloop, 0)

        drain_bytes(sem_c, const_drain_widths)
        drain_bytes(sem_b0, band_drain_widths)
        drain_bytes(sem_b1, band_drain_widths)

    return k


def kernel(inputs, table):
    S = inputs.shape[1]
    D = table.shape[1]
    out_t = _make_sc_kernel(S, D)(table.reshape(-1))
    return jnp.swapaxes(out_t, 1, 2)


# speedup vs baseline: 1.0004x; 1.0004x over previous
import functools

import jax
import jax.numpy as jnp
from jax import lax
from jax.experimental import pallas as pl
from jax.experimental.pallas import tpu as pltpu
from jax.experimental.pallas import tpu_sc as plsc

INPUT_DIM = 257
OUTPUT_DIM = 32
MAX_POS = (INPUT_DIM - 1) // 2


def _make_sc_kernel(S, D):
    NC, NS = 2, 16
    NW = NC * NS
    q_per_worker = S // NW
    n_vtiles = S // 128
    n_band = 3
    n_const = n_vtiles - n_band
    t0_max = n_vtiles - n_band

    mesh = plsc.VectorSubcoreMesh(
        core_axis_name="c", subcore_axis_name="s", num_cores=NC, num_subcores=NS
    )

    @functools.partial(
        pl.kernel,
        mesh=mesh,
        out_type=jax.ShapeDtypeStruct((S, D, S), jnp.float32),
        scratch_types=[
            pltpu.VMEM((INPUT_DIM * D,), jnp.float32),
            pltpu.VMEM((D, 2 * n_band * 128), jnp.float32),
            pltpu.VMEM_SHARED((D, 1024), jnp.float32),
            pltpu.VMEM_SHARED((D, 1024), jnp.float32),
            pltpu.SemaphoreType.DMA,
            pltpu.SemaphoreType.DMA,
            pltpu.SemaphoreType.DMA,
        ],
        compiler_params=pltpu.CompilerParams(
            use_tc_tiling_on_sc=True, needs_layout_passes=False
        ),
    )
    def k(table_hbm, out_hbm, table_v, band_v, cl_sh, cr_sh, sem_c, sem_b0, sem_b1):
        c = lax.axis_index("c")
        s = lax.axis_index("s")
        wid = s * NC + c
        q0 = wid * q_per_worker

        pltpu.sync_copy(table_hbm, table_v)
        lane = lax.iota(jnp.int32, 16)

        def drain_bytes(sem, widths):
            for w in widths:
                pltpu.make_async_copy(
                    out_hbm.at[0, :, pl.ds(0, w)],
                    cl_sh.at[:, pl.ds(0, w)],
                    sem,
                ).wait()

        const_drain_widths = [1024, 512, 128]
        band_drain_widths = [n_band * 128]

        def t0_of(start):
            return jnp.clip((S - MAX_POS - 128 - start) // 128, 0, t0_max)

        @pl.when(s == 0)
        def _():
            for d in range(D):
                left = plsc.load_gather(table_v, [jnp.full((16,), d, jnp.int32)])
                right = plsc.load_gather(
                    table_v, [jnp.full((16,), (INPUT_DIM - 1) * D + d, jnp.int32)]
                )
                for g in range(16):
                    band_v[d, pl.ds(16 * g, 16)] = left
                    band_v[d, pl.ds(256 + 16 * g, 16)] = right
            for rep in range(4):
                pltpu.sync_copy(
                    band_v.at[:, pl.ds(0, 256)], cl_sh.at[:, pl.ds(256 * rep, 256)]
                )
                pltpu.sync_copy(
                    band_v.at[:, pl.ds(256, 256)], cr_sh.at[:, pl.ds(256 * rep, 256)]
                )

        plsc.subcore_barrier()

        def qloop(i, carry):
            q = q0 + i
            start = (S - 1) - q
            t0 = t0_of(start)
            t0_prev = t0_of(start + 2)
            par = lax.rem(i, 2)
            boff = par * (n_band * 128)

            @pl.when(i >= 1)
            def _():
                drain_bytes(sem_c, const_drain_widths)

            issue_const_runs(q, t0)

            @pl.when(jnp.logical_and(i >= 2, par == 0))
            def _():
                drain_bytes(sem_b0, band_drain_widths)

            @pl.when(jnp.logical_and(i >= 2, par == 1))
            def _():
                drain_bytes(sem_b1, band_drain_widths)

            full = jnp.logical_or(i < 2, t0 != t0_prev)
            rel = S - 2 - start - 128 * t0
            g_lo = jnp.where(full, 0, jnp.clip((rel - MAX_POS) // 16, 0, n_band * 8))
            g_hi = jnp.where(
                full, n_band * 8, jnp.clip((rel + MAX_POS) // 16 + 1, 0, n_band * 8)
            )

            def gbody(g, carry2):
                kk = start + 128 * t0 + 16 * g + lane
                ids = jnp.clip(kk - (S - 1), -MAX_POS, MAX_POS) + MAX_POS
                base = ids * D
                for d in range(D):
                    band_v[d, pl.ds(boff + 16 * g, 16)] = plsc.load_gather(
                        table_v, [base + d]
                    )
                return carry2

            lax.fori_loop(g_lo, g_hi, gbody, 0)

            band_src = band_v.at[:, pl.ds(boff, n_band * 128)]

            @pl.when(par == 0)
            def _():
                pltpu.async_copy(
                    band_src, out_hbm.at[q, :, pl.ds(128 * t0, n_band * 128)], sem_b0
                )

            @pl.when(par == 1)
            def _():
                pltpu.async_copy(
                    band_src, out_hbm.at[q, :, pl.ds(128 * t0, n_band * 128)], sem_b1
                )

            return carry

        def issue_const_runs(q, t0):
            b8 = lax.rem(t0 // 8, 2)
            b4 = lax.rem(t0 // 4, 2)
            b2 = lax.rem(t0 // 2, 2)
            b1 = lax.rem(t0, 2)
            p1024 = 128 * t0 - 1024
            p512 = 128 * t0 - 1024 * b8 - 512
            p256 = 128 * t0 - 1024 * b8 - 512 * b4 - 256
            p128 = 128 * t0 - 1024 * b8 - 512 * b4 - 256 * b2 - 128

            @pl.when(b8 == 1)
            def _():
                pltpu.async_copy(
                    cl_sh, out_hbm.at[q, :, pl.ds(p1024, 1024)], sem_c
                )

            @pl.when(b4 == 1)
            def _():
                pltpu.async_copy(
                    cl_sh.at[:, pl.ds(0, 512)],
                    out_hbm.at[q, :, pl.ds(p512, 512)],
                    sem_c,
                )

            @pl.when(b2 == 1)
            def _():
                pltpu.async_copy(
                    cl_sh.at[:, pl.ds(0, 256)],
                    out_hbm.at[q, :, pl.ds(p256, 256)],
                    sem_c,
                )

            @pl.when(b1 == 1)
            def _():
                pltpu.async_copy(
                    cl_sh.at[:, pl.ds(0, 128)],
                    out_hbm.at[q, :, pl.ds(p128, 128)],
                    sem_c,
                )

            m = t0_max - t0
            m8 = lax.rem(m // 8, 2)
            m4 = lax.rem(m // 4, 2)
            m2 = lax.rem(m // 2, 2)
            m1 = lax.rem(m, 2)
            r0 = 128 * (t0 + n_band)
            r1024 = r0
            r512 = r0 + 1024 * m8
            r256 = r512 + 512 * m4
            r128 = r256 + 256 * m2

            @pl.when(m8 == 1)
            def _():
                pltpu.async_copy(
                    cr_sh, out_hbm.at[q, :, pl.ds(r1024, 1024)], sem_c
                )

            @pl.when(m4 == 1)
            def _():
                pltpu.async_copy(
                    cr_sh.at[:, pl.ds(0, 512)],
                    out_hbm.at[q, :, pl.ds(r512, 512)],
                    sem_c,
                )

            @pl.when(m2 == 1)
            def _():
                pltpu.async_copy(
                    cr_sh.at[:, pl.ds(0, 256)],
                    out_hbm.at[q, :, pl.ds(r256, 256)],
                    sem_c,
                )

            @pl.when(m1 == 1)
            def _():
                pltpu.async_copy(
                    cr_sh.at[:, pl.ds(0, 128)],
                    out_hbm.at[q, :, pl.ds(r128, 128)],
                    sem_c,
                )

        lax.fori_loop(0, q_per_worker, qloop, 0)

        drain_bytes(sem_c, const_drain_widths)
        drain_bytes(sem_b0, band_drain_widths)
        drain_bytes(sem_b1, band_drain_widths)

    return k


def kernel(inputs, table):
    S = inputs.shape[1]
    D = table.shape[1]
    out_t = _make_sc_kernel(S, D)(table.reshape(-1))
    return jnp.swapaxes(out_t, 1, 2)


# speedup vs baseline: 1.7260x; 1.7252x over previous
import functools

import jax
import jax.numpy as jnp
from jax import lax
from jax.experimental import pallas as pl
from jax.experimental.pallas import tpu as pltpu
from jax.experimental.pallas import tpu_sc as plsc

INPUT_DIM = 257
OUTPUT_DIM = 32
MAX_POS = (INPUT_DIM - 1) // 2


def _make_sc_kernel(S, D):
    NC, NS = 2, 16
    NW = NC * NS
    q_per_worker = S // NW
    n_vtiles = S // 128
    n_band = 3
    n_const = n_vtiles - n_band
    t0_max = n_vtiles - n_band

    mesh = plsc.VectorSubcoreMesh(
        core_axis_name="c", subcore_axis_name="s", num_cores=NC, num_subcores=NS
    )

    @functools.partial(
        pl.kernel,
        mesh=mesh,
        out_type=jax.ShapeDtypeStruct((S, D, S), jnp.float32),
        scratch_types=[
            pltpu.VMEM((INPUT_DIM * D,), jnp.float32),
            pltpu.VMEM((768 * D,), jnp.float32),
            pltpu.VMEM((D, 2 * n_band * 128), jnp.float32),
            pltpu.VMEM_SHARED((D, 1024), jnp.float32),
            pltpu.VMEM_SHARED((D, 1024), jnp.float32),
            pltpu.SemaphoreType.DMA,
            pltpu.SemaphoreType.DMA,
            pltpu.SemaphoreType.DMA,
        ],
        compiler_params=pltpu.CompilerParams(
            use_tc_tiling_on_sc=True, needs_layout_passes=False
        ),
    )
    def k(
        table_hbm, out_hbm, table_v, tt_v, band_v, cl_sh, cr_sh, sem_c, sem_b0, sem_b1
    ):
        c = lax.axis_index("c")
        s = lax.axis_index("s")
        wid = s * NC + c
        q0 = wid * q_per_worker

        pltpu.sync_copy(table_hbm, table_v)
        lane = lax.iota(jnp.int32, 16)

        PADL = 2 * MAX_POS
        base_k = (S - 1) - MAX_POS
        def tt_build(g, carry):
            ids = jnp.clip(g * 16 + lane - PADL, 0, INPUT_DIM - 1)
            flat = ids * D
            for d in range(D):
                tt_v[pl.ds(d * 768 + g * 16, 16)] = plsc.load_gather(
                    table_v, [flat + d]
                )
            return carry

        lax.fori_loop(0, 768 // 16, tt_build, 0)

        def drain_bytes(sem, widths):
            for w in widths:
                pltpu.make_async_copy(
                    out_hbm.at[0, :, pl.ds(0, w)],
                    cl_sh.at[:, pl.ds(0, w)],
                    sem,
                ).wait()

        const_drain_widths = [1024, 512, 128]
        band_drain_widths = [n_band * 128]

        def t0_of(start):
            return jnp.clip((S - MAX_POS - 128 - start) // 128, 0, t0_max)

        @pl.when(s == 0)
        def _():
            for d in range(D):
                left = plsc.load_gather(table_v, [jnp.full((16,), d, jnp.int32)])
                right = plsc.load_gather(
                    table_v, [jnp.full((16,), (INPUT_DIM - 1) * D + d, jnp.int32)]
                )
                for g in range(16):
                    band_v[d, pl.ds(16 * g, 16)] = left
                    band_v[d, pl.ds(256 + 16 * g, 16)] = right
            for rep in range(4):
                pltpu.sync_copy(
                    band_v.at[:, pl.ds(0, 256)], cl_sh.at[:, pl.ds(256 * rep, 256)]
                )
                pltpu.sync_copy(
                    band_v.at[:, pl.ds(256, 256)], cr_sh.at[:, pl.ds(256 * rep, 256)]
                )

        plsc.subcore_barrier()

        def qloop(i, carry):
            q = q0 + i
            start = (S - 1) - q
            t0 = t0_of(start)
            t0_prev = t0_of(start + 2)
            par = lax.rem(i, 2)
            boff = par * (n_band * 128)

            @pl.when(i >= 1)
            def _():
                drain_bytes(sem_c, const_drain_widths)

            issue_const_runs(q, t0)

            @pl.when(jnp.logical_and(i >= 2, par == 0))
            def _():
                drain_bytes(sem_b0, band_drain_widths)

            @pl.when(jnp.logical_and(i >= 2, par == 1))
            def _():
                drain_bytes(sem_b1, band_drain_widths)

            full = jnp.logical_or(i < 2, t0 != t0_prev)
            rel = S - 2 - start - 128 * t0
            g_lo = jnp.where(full, 0, jnp.clip((rel - MAX_POS) // 16, 0, n_band * 8))
            g_hi = jnp.where(
                full, n_band * 8, jnp.clip((rel + MAX_POS) // 16 + 1, 0, n_band * 8)
            )

            def gbody(g, carry2):
                m0 = start + 128 * t0 + 16 * g - base_k + PADL
                mvec = m0 + lane
                for d in range(D):
                    band_v[d, pl.ds(boff + 16 * g, 16)] = plsc.load_gather(
                        tt_v, [mvec + d * 768]
                    )
                return carry2

            lax.fori_loop(g_lo, g_hi, gbody, 0)

            band_src = band_v.at[:, pl.ds(boff, n_band * 128)]

            @pl.when(par == 0)
            def _():
                pltpu.async_copy(
                    band_src, out_hbm.at[q, :, pl.ds(128 * t0, n_band * 128)], sem_b0
                )

            @pl.when(par == 1)
            def _():
                pltpu.async_copy(
                    band_src, out_hbm.at[q, :, pl.ds(128 * t0, n_band * 128)], sem_b1
                )

            return carry

        def issue_const_runs(q, t0):
            b8 = lax.rem(t0 // 8, 2)
            b4 = lax.rem(t0 // 4, 2)
            b2 = lax.rem(t0 // 2, 2)
            b1 = lax.rem(t0, 2)
            p1024 = 128 * t0 - 1024
            p512 = 128 * t0 - 1024 * b8 - 512
            p256 = 128 * t0 - 1024 * b8 - 512 * b4 - 256
            p128 = 128 * t0 - 1024 * b8 - 512 * b4 - 256 * b2 - 128

            @pl.when(b8 == 1)
            def _():
                pltpu.async_copy(
                    cl_sh, out_hbm.at[q, :, pl.ds(p1024, 1024)], sem_c
                )

            @pl.when(b4 == 1)
            def _():
                pltpu.async_copy(
                    cl_sh.at[:, pl.ds(0, 512)],
                    out_hbm.at[q, :, pl.ds(p512, 512)],
                    sem_c,
                )

            @pl.when(b2 == 1)
            def _():
                pltpu.async_copy(
                    cl_sh.at[:, pl.ds(0, 256)],
                    out_hbm.at[q, :, pl.ds(p256, 256)],
                    sem_c,
                )

            @pl.when(b1 == 1)
            def _():
                pltpu.async_copy(
                    cl_sh.at[:, pl.ds(0, 128)],
                    out_hbm.at[q, :, pl.ds(p128, 128)],
                    sem_c,
                )

            m = t0_max - t0
            m8 = lax.rem(m // 8, 2)
            m4 = lax.rem(m // 4, 2)
            m2 = lax.rem(m // 2, 2)
            m1 = lax.rem(m, 2)
            r0 = 128 * (t0 + n_band)
            r1024 = r0
            r512 = r0 + 1024 * m8
            r256 = r512 + 512 * m4
            r128 = r256 + 256 * m2

            @pl.when(m8 == 1)
            def _():
                pltpu.async_copy(
                    cr_sh, out_hbm.at[q, :, pl.ds(r1024, 1024)], sem_c
                )

            @pl.when(m4 == 1)
            def _():
                pltpu.async_copy(
                    cr_sh.at[:, pl.ds(0, 512)],
                    out_hbm.at[q, :, pl.ds(r512, 512)],
                    sem_c,
                )

            @pl.when(m2 == 1)
            def _():
                pltpu.async_copy(
                    cr_sh.at[:, pl.ds(0, 256)],
                    out_hbm.at[q, :, pl.ds(r256, 256)],
                    sem_c,
                )

            @pl.when(m1 == 1)
            def _():
                pltpu.async_copy(
                    cr_sh.at[:, pl.ds(0, 128)],
                    out_hbm.at[q, :, pl.ds(r128, 128)],
                    sem_c,
                )

        lax.fori_loop(0, q_per_worker, qloop, 0)

        drain_bytes(sem_c, const_drain_widths)
        drain_bytes(sem_b0, band_drain_widths)
        drain_bytes(sem_b1, band_drain_widths)

    return k


def kernel(inputs, table):
    S = inputs.shape[1]
    D = table.shape[1]
    out_t = _make_sc_kernel(S, D)(table.reshape(-1))
    return jnp.swapaxes(out_t, 1, 2)


# speedup vs baseline: 1.7972x; 1.0412x over previous
import functools

import jax
import jax.numpy as jnp
from jax import lax
from jax.experimental import pallas as pl
from jax.experimental.pallas import tpu as pltpu
from jax.experimental.pallas import tpu_sc as plsc

INPUT_DIM = 257
OUTPUT_DIM = 32
MAX_POS = (INPUT_DIM - 1) // 2


def _make_sc_kernel(S, D):
    NC, NS = 2, 16
    NW = NC * NS
    q_per_worker = S // NW
    n_vtiles = S // 128
    n_band = 3
    n_const = n_vtiles - n_band
    t0_max = n_vtiles - n_band

    mesh = plsc.VectorSubcoreMesh(
        core_axis_name="c", subcore_axis_name="s", num_cores=NC, num_subcores=NS
    )

    @functools.partial(
        pl.kernel,
        mesh=mesh,
        out_type=jax.ShapeDtypeStruct((S, D, S), jnp.float32),
        scratch_types=[
            pltpu.VMEM((INPUT_DIM * D,), jnp.float32),
            pltpu.VMEM((768 * D,), jnp.float32),
            pltpu.VMEM((2 * 768,), jnp.float32),
            pltpu.VMEM_SHARED((768 * D,), jnp.float32),
            pltpu.VMEM((D, 2 * n_band * 128), jnp.float32),
            pltpu.VMEM_SHARED((D, 1024), jnp.float32),
            pltpu.VMEM_SHARED((D, 1024), jnp.float32),
            pltpu.SemaphoreType.DMA,
            pltpu.SemaphoreType.DMA,
            pltpu.SemaphoreType.DMA,
        ],
        compiler_params=pltpu.CompilerParams(
            use_tc_tiling_on_sc=True, needs_layout_passes=False
        ),
    )
    def k(
        table_hbm,
        out_hbm,
        table_v,
        tt_v,
        stage_v,
        tt_sh,
        band_v,
        cl_sh,
        cr_sh,
        sem_c,
        sem_b0,
        sem_b1,
    ):
        c = lax.axis_index("c")
        s = lax.axis_index("s")
        wid = s * NC + c
        q0 = wid * q_per_worker

        pltpu.sync_copy(table_hbm, table_v)
        lane = lax.iota(jnp.int32, 16)

        PADL = 2 * MAX_POS
        base_k = (S - 1) - MAX_POS
        def tt_build(g, carry):
            ids = jnp.clip(g * 16 + lane - PADL, 0, INPUT_DIM - 1)
            flat = ids * D
            for dd in range(2):
                stage_v[pl.ds(dd * 768 + g * 16, 16)] = plsc.load_gather(
                    table_v, [flat + (2 * s + dd)]
                )
            return carry

        lax.fori_loop(0, 768 // 16, tt_build, 0)
        pltpu.sync_copy(stage_v, tt_sh.at[pl.ds(s * (2 * 768), 2 * 768)])

        def drain_bytes(sem, widths):
            for w in widths:
                pltpu.make_async_copy(
                    out_hbm.at[0, :, pl.ds(0, w)],
                    cl_sh.at[:, pl.ds(0, w)],
                    sem,
                ).wait()

        const_drain_widths = [1024, 512, 128]
        band_drain_widths = [n_band * 128]

        def t0_of(start):
            return jnp.clip((S - MAX_POS - 128 - start) // 128, 0, t0_max)

        @pl.when(s == 0)
        def _():
            for d in range(D):
                left = plsc.load_gather(table_v, [jnp.full((16,), d, jnp.int32)])
                right = plsc.load_gather(
                    table_v, [jnp.full((16,), (INPUT_DIM - 1) * D + d, jnp.int32)]
                )
                for g in range(16):
                    band_v[d, pl.ds(16 * g, 16)] = left
                    band_v[d, pl.ds(256 + 16 * g, 16)] = right
            for rep in range(4):
                pltpu.sync_copy(
                    band_v.at[:, pl.ds(0, 256)], cl_sh.at[:, pl.ds(256 * rep, 256)]
                )
                pltpu.sync_copy(
                    band_v.at[:, pl.ds(256, 256)], cr_sh.at[:, pl.ds(256 * rep, 256)]
                )

        plsc.subcore_barrier()
        pltpu.sync_copy(tt_sh, tt_v)

        def qloop(i, carry):
            q = q0 + i
            start = (S - 1) - q
            t0 = t0_of(start)
            t0_prev = t0_of(start + 2)
            par = lax.rem(i, 2)
            boff = par * (n_band * 128)

            @pl.when(i >= 1)
            def _():
                drain_bytes(sem_c, const_drain_widths)

            issue_const_runs(q, t0)

            @pl.when(jnp.logical_and(i >= 2, par == 0))
            def _():
                drain_bytes(sem_b0, band_drain_widths)

            @pl.when(jnp.logical_and(i >= 2, par == 1))
            def _():
                drain_bytes(sem_b1, band_drain_widths)

            full = jnp.logical_or(i < 2, t0 != t0_prev)
            rel = S - 2 - start - 128 * t0
            g_lo = jnp.where(full, 0, jnp.clip((rel - MAX_POS) // 16, 0, n_band * 8))
            g_hi = jnp.where(
                full, n_band * 8, jnp.clip((rel + MAX_POS) // 16 + 1, 0, n_band * 8)
            )

            def gbody(g, carry2):
                m0 = start + 128 * t0 + 16 * g - base_k + PADL
                mvec = m0 + lane
                for d in range(D):
                    band_v[d, pl.ds(boff + 16 * g, 16)] = plsc.load_gather(
                        tt_v, [mvec + d * 768]
                    )
                return carry2

            lax.fori_loop(g_lo, g_hi, gbody, 0)

            band_src = band_v.at[:, pl.ds(boff, n_band * 128)]

            @pl.when(par == 0)
            def _():
                pltpu.async_copy(
                    band_src, out_hbm.at[q, :, pl.ds(128 * t0, n_band * 128)], sem_b0
                )

            @pl.when(par == 1)
            def _():
                pltpu.async_copy(
                    band_src, out_hbm.at[q, :, pl.ds(128 * t0, n_band * 128)], sem_b1
                )

            return carry

        def issue_const_runs(q, t0):
            b8 = lax.rem(t0 // 8, 2)
            b4 = lax.rem(t0 // 4, 2)
            b2 = lax.rem(t0 // 2, 2)
            b1 = lax.rem(t0, 2)
            p1024 = 128 * t0 - 1024
            p512 = 128 * t0 - 1024 * b8 - 512
            p256 = 128 * t0 - 1024 * b8 - 512 * b4 - 256
            p128 = 128 * t0 - 1024 * b8 - 512 * b4 - 256 * b2 - 128

            @pl.when(b8 == 1)
            def _():
                pltpu.async_copy(
                    cl_sh, out_hbm.at[q, :, pl.ds(p1024, 1024)], sem_c
                )

            @pl.when(b4 == 1)
            def _():
                pltpu.async_copy(
                    cl_sh.at[:, pl.ds(0, 512)],
                    out_hbm.at[q, :, pl.ds(p512, 512)],
                    sem_c,
                )

            @pl.when(b2 == 1)
            def _():
                pltpu.async_copy(
                    cl_sh.at[:, pl.ds(0, 256)],
                    out_hbm.at[q, :, pl.ds(p256, 256)],
                    sem_c,
                )

            @pl.when(b1 == 1)
            def _():
                pltpu.async_copy(
                    cl_sh.at[:, pl.ds(0, 128)],
                    out_hbm.at[q, :, pl.ds(p128, 128)],
                    sem_c,
                )

            m = t0_max - t0
            m8 = lax.rem(m // 8, 2)
            m4 = lax.rem(m // 4, 2)
            m2 = lax.rem(m // 2, 2)
            m1 = lax.rem(m, 2)
            r0 = 128 * (t0 + n_band)
            r1024 = r0
            r512 = r0 + 1024 * m8
            r256 = r512 + 512 * m4
            r128 = r256 + 256 * m2

            @pl.when(m8 == 1)
            def _():
                pltpu.async_copy(
                    cr_sh, out_hbm.at[q, :, pl.ds(r1024, 1024)], sem_c
                )

            @pl.when(m4 == 1)
            def _():
                pltpu.async_copy(
                    cr_sh.at[:, pl.ds(0, 512)],
                    out_hbm.at[q, :, pl.ds(r512, 512)],
                    sem_c,
                )

            @pl.when(m2 == 1)
            def _():
                pltpu.async_copy(
                    cr_sh.at[:, pl.ds(0, 256)],
                    out_hbm.at[q, :, pl.ds(r256, 256)],
                    sem_c,
                )

            @pl.when(m1 == 1)
            def _():
                pltpu.async_copy(
                    cr_sh.at[:, pl.ds(0, 128)],
                    out_hbm.at[q, :, pl.ds(r128, 128)],
                    sem_c,
                )

        lax.fori_loop(0, q_per_worker, qloop, 0)

        drain_bytes(sem_c, const_drain_widths)
        drain_bytes(sem_b0, band_drain_widths)
        drain_bytes(sem_b1, band_drain_widths)

    return k


def kernel(inputs, table):
    S = inputs.shape[1]
    D = table.shape[1]
    out_t = _make_sc_kernel(S, D)(table.reshape(-1))
    return jnp.swapaxes(out_t, 1, 2)
